# baseline (speedup 1.0000x reference)
import functools
import math

import jax
import jax.numpy as jnp
from jax import lax
from jax.experimental import pallas as pl
from jax.experimental.pallas import tpu as pltpu
from jax.experimental.pallas import tpu_sc as plsc

_PC = 4


def _make_emb_kernel(P, Q, D, NC, NS):
    NW = NC * NS
    QT = Q // 128
    assert QT == NW
    DT = D // 8
    tok_per_w = 128 * P
    row_out = DT * QT * 8 * 128
    run = 8 * 128
    NCH = P // _PC
    assert NCH % 2 == 0
    mesh = plsc.VectorSubcoreMesh(core_axis_name="c", subcore_axis_name="s")
    scale = math.sqrt(D)

    @functools.partial(
        pl.kernel,
        mesh=mesh,
        compiler_params=pltpu.CompilerParams(
            use_tc_tiling_on_sc=False,
            needs_layout_passes=False,
            disable_bounds_checks=True,
        ),
        out_type=jax.ShapeDtypeStruct((P, row_out), jnp.float32),
        scratch_types=[
            pltpu.VMEM((tok_per_w,), jnp.int32),
            pltpu.VMEM((2, _PC * 128), jnp.int32),
            pltpu.VMEM((2, _PC * 128, D), jnp.float32),
            pltpu.VMEM((2, _PC, D * 128), jnp.float32),
            [pltpu.SemaphoreType.DMA] * 2,
            [pltpu.SemaphoreType.DMA] * 2,
        ],
    )
    def emb(ids_hbm, table_hbm, out_hbm, idsb, idx_v, rows_v, tp_v, gsem, wsem):
        w = lax.axis_index("s") * NC + lax.axis_index("c")

        pltpu.sync_copy(ids_hbm.at[pl.ds(w * tok_per_w, tok_per_w)], idsb)

        iota = lax.iota(jnp.int32, 16)
        iotaP = iota * P
        iota128 = iota * 128
        iotaD = iota * D

        def build_idx_and_gather(c, b):
            for h in range(_PC):
                for j in range(8):
                    vals = plsc.load_gather(
                        idsb, [iotaP + (16 * j * P + (_PC * c + h))]
                    )
                    idx_v[b, pl.ds(h * 128 + 16 * j, 16)] = vals
            pltpu.async_copy(table_hbm.at[idx_v.at[b]], rows_v.at[b], gsem[b])

        def wait_gather(b):
            pltpu.make_async_copy(
                table_hbm.at[idx_v.at[b]], rows_v.at[b], gsem[b]
            ).wait()

        def transpose_scale(b):
            dconsts = [jnp.full((16,), d, jnp.int32) for d in range(D)]
            for h in range(_PC):
                tpb = tp_v.at[b, h]
                rvb = rows_v.at[b]

                @plsc.parallel_loop(0, 8, unroll=2)
                def _(g):
                    a0 = iota + (h * 128 + 16 * g)
                    for d in range(D):
                        v = plsc.load_gather(rvb, [a0, dconsts[d]]) * scale
                        tpb[pl.ds(d * 128 + 16 * g, 16)] = v

        def issue_writeback(c, b):
            for h in range(_PC):
                for r in range(DT):
                    pltpu.async_copy(
                        tp_v.at[b, h, pl.ds(r * run, run)],
                        out_hbm.at[
                            _PC * c + h, pl.ds(r * (QT * run) + w * run, run)
                        ],
                        wsem[b],
                    )

        def wait_writeback(b):
            pltpu.make_async_copy(
                tp_v.at[b], out_hbm.at[0, pl.ds(0, _PC * D * 128)], wsem[b]
            ).wait()

        build_idx_and_gather(0, 0)

        @pl.loop(0, NCH, step=2)
        def _(co):
            for b in range(2):
                c = co + b
                wait_gather(b)
                if b == 0:
                    build_idx_and_gather(c + 1, 1)
                else:

                    @pl.when(co < NCH - 2)
                    def _():
                        build_idx_and_gather(c + 1, 0)

                @pl.when(co > 0)
                def _():
                    wait_writeback(b)

                transpose_scale(b)
                issue_writeback(c, b)

        wait_writeback(0)
        wait_writeback(1)

    return emb


def kernel(token_ids, embedding_weight):
    Q, P = token_ids.shape
    V, D = embedding_weight.shape
    info = plsc.get_sparse_core_info()
    NC, NS = info.num_cores, info.num_subcores
    NW = NC * NS
    flat_ids = token_ids.reshape(Q * P).astype(jnp.int32)
    emb = _make_emb_kernel(P, Q, D, NC, NS)
    out2 = emb(flat_ids, embedding_weight)
    out5 = out2.reshape(P, D // 8, NW, 8, 128)
    return out5.transpose(2, 4, 0, 1, 3).reshape(Q, P, D)


# speedup vs baseline: 1.1104x; 1.1104x over previous
import functools
import math

import jax
import jax.numpy as jnp
from jax import lax
from jax.experimental import pallas as pl
from jax.experimental.pallas import tpu as pltpu
from jax.experimental.pallas import tpu_sc as plsc

_PC = 4


def _make_emb_kernel(P, Q, D, NC, NS):
    NW = NC * NS
    QT = Q // 128
    assert QT == NW
    DT = D // 8
    tok_per_w = 128 * P
    row_out = DT * QT * 8 * 128
    run = 8 * 128
    NCH = P // _PC
    assert NCH % 2 == 0
    mesh = plsc.VectorSubcoreMesh(core_axis_name="c", subcore_axis_name="s")
    scale = math.sqrt(D)

    @functools.partial(
        pl.kernel,
        mesh=mesh,
        compiler_params=pltpu.CompilerParams(
            use_tc_tiling_on_sc=False,
            needs_layout_passes=False,
            disable_bounds_checks=True,
        ),
        out_type=jax.ShapeDtypeStruct((P, row_out), jnp.float32),
        scratch_types=[
            pltpu.VMEM((tok_per_w,), jnp.int32),
            pltpu.VMEM((2, _PC * 128), jnp.int32),
            pltpu.VMEM((2, _PC * 128, D), jnp.float32),
            pltpu.VMEM((2, _PC, D * 128), jnp.float32),
            [pltpu.SemaphoreType.DMA] * 2,
            [pltpu.SemaphoreType.DMA] * 2,
        ],
    )
    def emb(ids_hbm, table_hbm, out_hbm, idsb, idx_v, rows_v, tp_v, gsem, wsem):
        w = lax.axis_index("s") * NC + lax.axis_index("c")

        pltpu.sync_copy(ids_hbm.at[pl.ds(w * tok_per_w, tok_per_w)], idsb)

        iota = lax.iota(jnp.int32, 16)
        iotaP = iota * P
        iota128 = iota * 128

        def build_idx_and_gather(c, b):
            for h in range(_PC):
                for j in range(8):
                    vals = plsc.load_gather(
                        idsb, [iotaP + (16 * j * P + (_PC * c + h))]
                    )
                    idx_v[b, pl.ds(h * 128 + 16 * j, 16)] = vals
            pltpu.async_copy(table_hbm.at[idx_v.at[b]], rows_v.at[b], gsem[b])

        def wait_gather(b):
            pltpu.make_async_copy(
                table_hbm.at[idx_v.at[b]], rows_v.at[b], gsem[b]
            ).wait()

        def transpose_scale(b):
            for h in range(_PC):
                tpb = tp_v.at[b, h]

                @plsc.parallel_loop(0, 128, unroll=8, carry=iota128)
                def _(t, a0):
                    for j in range(2):
                        v = rows_v[b, h * 128 + t, pl.ds(16 * j, 16)] * scale
                        plsc.store_scatter(tpb, [a0 + (2048 * j)], v)
                    return a0 + 1

        def issue_writeback(c, b):
            for h in range(_PC):
                for r in range(DT):
                    pltpu.async_copy(
                        tp_v.at[b, h, pl.ds(r * run, run)],
                        out_hbm.at[
                            _PC * c + h, pl.ds(r * (QT * run) + w * run, run)
                        ],
                        wsem[b],
                    )

        def wait_writeback(b):
            pltpu.make_async_copy(
                tp_v.at[b], out_hbm.at[0, pl.ds(0, _PC * D * 128)], wsem[b]
            ).wait()

        build_idx_and_gather(0, 0)

        @pl.loop(0, NCH, step=2)
        def _(co):
            for b in range(2):
                c = co + b
                wait_gather(b)
                if b == 0:
                    build_idx_and_gather(c + 1, 1)
                else:

                    @pl.when(co < NCH - 2)
                    def _():
                        build_idx_and_gather(c + 1, 0)

                @pl.when(co > 0)
                def _():
                    wait_writeback(b)

                transpose_scale(b)
                issue_writeback(c, b)

        wait_writeback(0)
        wait_writeback(1)

    return emb


def kernel(token_ids, embedding_weight):
    Q, P = token_ids.shape
    V, D = embedding_weight.shape
    info = plsc.get_sparse_core_info()
    NC, NS = info.num_cores, info.num_subcores
    NW = NC * NS
    flat_ids = token_ids.reshape(Q * P).astype(jnp.int32)
    emb = _make_emb_kernel(P, Q, D, NC, NS)
    out2 = emb(flat_ids, embedding_weight)
    out5 = out2.reshape(P, D // 8, NW, 8, 128)
    return out5.transpose(2, 4, 0, 1, 3).reshape(Q, P, D)


# speedup vs baseline: 1.1126x; 1.0020x over previous
import functools
import math

import jax
import jax.numpy as jnp
from jax import lax
from jax.experimental import pallas as pl
from jax.experimental.pallas import tpu as pltpu
from jax.experimental.pallas import tpu_sc as plsc

_PC = 4


def _make_emb_kernel(P, Q, D, NC, NS):
    NW = NC * NS
    QT = Q // 128
    assert QT == NW
    DT = D // 8
    tok_per_w = 128 * P
    row_out = DT * QT * 8 * 128
    run = 8 * 128
    NCH = P // _PC
    assert NCH % 2 == 0
    mesh = plsc.VectorSubcoreMesh(core_axis_name="c", subcore_axis_name="s")
    scale = math.sqrt(D)

    @functools.partial(
        pl.kernel,
        mesh=mesh,
        compiler_params=pltpu.CompilerParams(
            use_tc_tiling_on_sc=False,
            needs_layout_passes=False,
            disable_bounds_checks=True,
        ),
        out_type=jax.ShapeDtypeStruct((P, row_out), jnp.float32),
        scratch_types=[
            pltpu.VMEM((tok_per_w,), jnp.int32),
            pltpu.VMEM((2, _PC * 128), jnp.int32),
            pltpu.VMEM((2, _PC * 128, D), jnp.float32),
            pltpu.VMEM((2, _PC, D * 128), jnp.float32),
            [pltpu.SemaphoreType.DMA] * 2,
            [pltpu.SemaphoreType.DMA] * 2,
        ],
    )
    def emb(ids_hbm, table_hbm, out_hbm, idsb, idx_v, rows_v, tp_v, gsem, wsem):
        w = lax.axis_index("s") * NC + lax.axis_index("c")

        pltpu.sync_copy(ids_hbm.at[pl.ds(w * tok_per_w, tok_per_w)], idsb)

        iota = lax.iota(jnp.int32, 16)
        iotaP = iota * P
        iota128 = iota * 128

        def build_idx_and_gather(c, b):
            for h in range(_PC):
                for j in range(8):
                    vals = plsc.load_gather(
                        idsb, [iotaP + (16 * j * P + (_PC * c + h))]
                    )
                    idx_v[b, pl.ds(h * 128 + 16 * j, 16)] = vals
            pltpu.async_copy(table_hbm.at[idx_v.at[b]], rows_v.at[b], gsem[b])

        def wait_gather(b):
            pltpu.make_async_copy(
                table_hbm.at[idx_v.at[b]], rows_v.at[b], gsem[b]
            ).wait()

        def transpose_scale(b):
            for h in range(_PC):
                tpb = tp_v.at[b, h]

                for j in range(2):

                    @plsc.parallel_loop(0, 128, unroll=8, carry=iota128 + 2048 * j)
                    def _(t, a0):
                        v = rows_v[b, h * 128 + t, pl.ds(16 * j, 16)] * scale
                        plsc.store_scatter(tpb, [a0], v)
                        return a0 + 1

        def issue_writeback(c, b):
            for h in range(_PC):
                for r in range(DT):
                    pltpu.async_copy(
                        tp_v.at[b, h, pl.ds(r * run, run)],
                        out_hbm.at[
                            _PC * c + h, pl.ds(r * (QT * run) + w * run, run)
                        ],
                        wsem[b],
                    )

        def wait_writeback(b):
            pltpu.make_async_copy(
                tp_v.at[b], out_hbm.at[0, pl.ds(0, _PC * D * 128)], wsem[b]
            ).wait()

        build_idx_and_gather(0, 0)

        @pl.loop(0, NCH, step=2)
        def _(co):
            for b in range(2):
                c = co + b
                wait_gather(b)
                if b == 0:
                    build_idx_and_gather(c + 1, 1)
                else:

                    @pl.when(co < NCH - 2)
                    def _():
                        build_idx_and_gather(c + 1, 0)

                @pl.when(co > 0)
                def _():
                    wait_writeback(b)

                transpose_scale(b)
                issue_writeback(c, b)

        wait_writeback(0)
        wait_writeback(1)

    return emb


def kernel(token_ids, embedding_weight):
    Q, P = token_ids.shape
    V, D = embedding_weight.shape
    info = plsc.get_sparse_core_info()
    NC, NS = info.num_cores, info.num_subcores
    NW = NC * NS
    flat_ids = token_ids.reshape(Q * P).astype(jnp.int32)
    emb = _make_emb_kernel(P, Q, D, NC, NS)
    out2 = emb(flat_ids, embedding_weight)
    out5 = out2.reshape(P, D // 8, NW, 8, 128)
    return out5.transpose(2, 4, 0, 1, 3).reshape(Q, P, D)
